# baseline (speedup 1.0000x reference)
import jax
import jax.numpy as jnp
from jax.experimental import pallas as pl
from jax.experimental.pallas import tpu as pltpu

_B, _C, _S = 256, 768, 196
_NC = 10
_SB = 7
_NSTEP = _S // _SB
_BH = _B // 2


def _body(f_ref, w_ref, b_ref, o_ref, acc_ref):
    i = pl.program_id(1)
    partial = jnp.sum(f_ref[...], axis=0)

    @pl.when(i == 0)
    def _init():
        acc_ref[...] = partial

    @pl.when(i > 0)
    def _acc():
        acc_ref[...] += partial

    @pl.when(i == _NSTEP - 1)
    def _fin():
        pooled = acc_ref[...] * (1.0 / _S)
        o_ref[...] = jax.lax.dot_general(
            pooled, w_ref[...], (((2,), (1,)), ((), ())),
            preferred_element_type=jnp.float32) + b_ref[...]


def kernel(features, W, b):
    f4 = features.transpose(2, 3, 0, 1).reshape(_S, 2, _BH, _C)
    out = pl.pallas_call(
        _body,
        grid=(2, _NSTEP),
        in_specs=[
            pl.BlockSpec((_SB, 1, _BH, _C), lambda h, i: (i, h, 0, 0)),
            pl.BlockSpec((_NC, _C), lambda h, i: (0, 0)),
            pl.BlockSpec((1, _NC), lambda h, i: (0, 0)),
        ],
        out_specs=pl.BlockSpec((1, _BH, _NC), lambda h, i: (h, 0, 0)),
        out_shape=jax.ShapeDtypeStruct((2, _BH, _NC), jnp.float32),
        scratch_shapes=[pltpu.VMEM((1, _BH, _C), jnp.float32)],
    )(f4, W, b.reshape(1, _NC))
    return out.reshape(_B, _NC)


# speedup vs baseline: 1.3102x; 1.3102x over previous
import jax
import jax.numpy as jnp
from jax.experimental import pallas as pl
from jax.experimental.pallas import tpu as pltpu

_B, _C, _S = 256, 768, 196
_NC = 10
_SB = 7
_NSTEP = _S // _SB


def _body(f_ref, w_ref, b_ref, o_ref, acc_ref):
    i = pl.program_id(0)
    partial = jnp.sum(f_ref[...], axis=0)

    @pl.when(i == 0)
    def _init():
        acc_ref[...] = partial

    @pl.when(i > 0)
    def _acc():
        acc_ref[...] += partial

    @pl.when(i == _NSTEP - 1)
    def _fin():
        pooled = acc_ref[...] * (1.0 / _S)
        o_ref[...] = jax.lax.dot_general(
            pooled, w_ref[...], (((1,), (1,)), ((), ())),
            preferred_element_type=jnp.float32) + b_ref[...]


def kernel(features, W, b):
    f3 = features.transpose(2, 3, 0, 1).reshape(_S, _B, _C)
    out = pl.pallas_call(
        _body,
        grid=(_NSTEP,),
        in_specs=[
            pl.BlockSpec((_SB, _B, _C), lambda i: (i, 0, 0)),
            pl.BlockSpec((_NC, _C), lambda i: (0, 0)),
            pl.BlockSpec((1, _NC), lambda i: (0, 0)),
        ],
        out_specs=pl.BlockSpec((_B, _NC), lambda i: (0, 0)),
        out_shape=jax.ShapeDtypeStruct((_B, _NC), jnp.float32),
        scratch_shapes=[pltpu.VMEM((_B, _C), jnp.float32)],
    )(f3, W, b.reshape(1, _NC))
    return out
